# baseline (speedup 1.0000x reference)
import jax
import jax.numpy as jnp
from jax import lax
from jax.experimental import pallas as pl
from jax.experimental.pallas import tpu as pltpu
from jax.experimental.pallas import tpu_sc as plsc

BATCH = 16384
EMBED_DIM = 32
NUM_USERS = 1000000
NUM_MISSIONS = 100000
NUM_CORES = 2
LANES = 16
NUM_WORKERS = NUM_CORES * 16
B_PER_W = BATCH // NUM_WORKERS
GROUPS = B_PER_W // LANES


def _gather_dims_transposed(flat_hbm, idx_v, n_rows, vals_v, a_v, b_v, sem):
    def seed_body(g, carry):
        off = g * LANES
        idx = idx_v[pl.ds(off, LANES)]
        a_v[pl.ds(off, LANES)] = idx
        b_v[pl.ds(off, LANES)] = idx + n_rows
        return carry

    lax.fori_loop(0, GROUPS, seed_body, 0)

    bufs = (a_v, b_v)
    copies = []
    for d in range(EMBED_DIM):
        buf = bufs[d % 2]
        copies.append(pltpu.async_copy(flat_hbm.at[buf], vals_v.at[d], sem))
        if d >= 1:
            copies[d - 1].wait()
            if d + 1 < EMBED_DIM:
                prev = bufs[(d - 1) % 2]

                def bump_body(g, carry, prev=prev):
                    off = g * LANES
                    prev[pl.ds(off, LANES)] = (
                        prev[pl.ds(off, LANES)] + 2 * n_rows)
                    return carry

                lax.fori_loop(0, GROUPS, bump_body, 0)
    copies[EMBED_DIM - 1].wait()


def _mf_kernel(user_hbm, mission_hbm, uflat_hbm, mflat_hbm, ubias_hbm,
               mbias_hbm, bias_hbm, out_hbm,
               uidx_v, midx_v, a_v, b_v, uvals_v, mvals_v, ub_v, mb_v,
               bias_v, out_v, sem_g, sem_ub, sem_mb):
    wid = lax.axis_index("s") * NUM_CORES + lax.axis_index("c")
    base = wid * B_PER_W

    bias_v[...] = jnp.zeros((LANES,), jnp.float32)
    zidx = uidx_v
    zidx[pl.ds(0, LANES)] = jnp.zeros((LANES,), jnp.int32)
    pltpu.sync_copy(bias_hbm.at[zidx.at[pl.ds(0, LANES)]], bias_v)
    bias_vec = bias_v[...]

    pltpu.sync_copy(user_hbm.at[pl.ds(base, B_PER_W)], uidx_v)
    pltpu.sync_copy(mission_hbm.at[pl.ds(base, B_PER_W)], midx_v)

    cp_ub = pltpu.async_copy(ubias_hbm.at[uidx_v], ub_v, sem_ub)
    cp_mb = pltpu.async_copy(mbias_hbm.at[midx_v], mb_v, sem_mb)

    _gather_dims_transposed(uflat_hbm, uidx_v, NUM_USERS, uvals_v,
                            a_v, b_v, sem_g)
    _gather_dims_transposed(mflat_hbm, midx_v, NUM_MISSIONS, mvals_v,
                            a_v, b_v, sem_g)
    cp_ub.wait()
    cp_mb.wait()

    def group_body(g, carry):
        off = g * LANES
        acc = ub_v[pl.ds(off, LANES)] + mb_v[pl.ds(off, LANES)] + bias_vec
        for d in range(EMBED_DIM):
            acc = acc + (uvals_v[d, pl.ds(off, LANES)]
                         * mvals_v[d, pl.ds(off, LANES)])
        out_v[pl.ds(off, LANES)] = acc
        return carry

    lax.fori_loop(0, GROUPS, group_body, 0)

    pltpu.sync_copy(out_v, out_hbm.at[pl.ds(base, B_PER_W)])


@jax.jit
def _run(user, mission, uflat, mflat, ubias, mbias, bias):
    mesh = plsc.VectorSubcoreMesh(core_axis_name="c", subcore_axis_name="s")
    kfn = pl.kernel(
        _mf_kernel,
        out_type=jax.ShapeDtypeStruct((BATCH,), jnp.float32),
        mesh=mesh,
        compiler_params=pltpu.CompilerParams(needs_layout_passes=False,
                                             use_tc_tiling_on_sc=False),
        scratch_types=[
            pltpu.VMEM((B_PER_W,), jnp.int32),
            pltpu.VMEM((B_PER_W,), jnp.int32),
            pltpu.VMEM((B_PER_W,), jnp.int32),
            pltpu.VMEM((B_PER_W,), jnp.int32),
            pltpu.VMEM((EMBED_DIM, B_PER_W), jnp.float32),
            pltpu.VMEM((EMBED_DIM, B_PER_W), jnp.float32),
            pltpu.VMEM((B_PER_W,), jnp.float32),
            pltpu.VMEM((B_PER_W,), jnp.float32),
            pltpu.VMEM((LANES,), jnp.float32),
            pltpu.VMEM((B_PER_W,), jnp.float32),
            pltpu.SemaphoreType.DMA,
            pltpu.SemaphoreType.DMA,
            pltpu.SemaphoreType.DMA,
        ],
    )
    return kfn(user, mission, uflat, mflat, ubias, mbias, bias)


def kernel(user, mission, user_embedding, mission_embedding, user_bias,
           mission_bias, bias):
    user = user.astype(jnp.int32)
    mission = mission.astype(jnp.int32)
    return _run(user, mission,
                user_embedding.T.reshape(-1), mission_embedding.T.reshape(-1),
                user_bias.reshape(-1), mission_bias.reshape(-1),
                bias.reshape(-1))


# speedup vs baseline: 1.0110x; 1.0110x over previous
import jax
import jax.numpy as jnp
from jax import lax
from jax.experimental import pallas as pl
from jax.experimental.pallas import tpu as pltpu
from jax.experimental.pallas import tpu_sc as plsc

BATCH = 16384
EMBED_DIM = 32
NUM_USERS = 1000000
NUM_MISSIONS = 100000
NUM_CORES = 2
LANES = 16
NUM_WORKERS = NUM_CORES * 16
B_PER_W = BATCH // NUM_WORKERS
GROUPS = B_PER_W // LANES


def _seed_indices(idx_v, n_rows, idxall_v):
    def seed_body(g, carry):
        off = g * LANES
        idx = idx_v[pl.ds(off, LANES)]
        for d in range(EMBED_DIM):
            idxall_v[d, pl.ds(off, LANES)] = idx + d * n_rows
        return carry

    lax.fori_loop(0, GROUPS, seed_body, 0)


def _fire_dim_gathers(flat_hbm, idxall_v, vals_v, sem):
    return [
        pltpu.async_copy(flat_hbm.at[idxall_v.at[d]], vals_v.at[d], sem)
        for d in range(EMBED_DIM)
    ]


def _mf_kernel(user_hbm, mission_hbm, uflat_hbm, mflat_hbm, ubias_hbm,
               mbias_hbm, bias_hbm, out_hbm,
               uidx_v, midx_v, uidxall_v, midxall_v, uvals_v, mvals_v,
               ub_v, mb_v, bias_v, out_v, sem_u, sem_m, sem_ub, sem_mb):
    wid = lax.axis_index("s") * NUM_CORES + lax.axis_index("c")
    base = wid * B_PER_W

    bias_v[...] = jnp.zeros((LANES,), jnp.float32)
    zidx = uidx_v
    zidx[pl.ds(0, LANES)] = jnp.zeros((LANES,), jnp.int32)
    pltpu.sync_copy(bias_hbm.at[zidx.at[pl.ds(0, LANES)]], bias_v)
    bias_vec = bias_v[...]

    pltpu.sync_copy(user_hbm.at[pl.ds(base, B_PER_W)], uidx_v)
    pltpu.sync_copy(mission_hbm.at[pl.ds(base, B_PER_W)], midx_v)

    cp_ub = pltpu.async_copy(ubias_hbm.at[uidx_v], ub_v, sem_ub)
    cp_mb = pltpu.async_copy(mbias_hbm.at[midx_v], mb_v, sem_mb)

    _seed_indices(uidx_v, NUM_USERS, uidxall_v)
    _seed_indices(midx_v, NUM_MISSIONS, midxall_v)
    ucopies = _fire_dim_gathers(uflat_hbm, uidxall_v, uvals_v, sem_u)
    mcopies = _fire_dim_gathers(mflat_hbm, midxall_v, mvals_v, sem_m)
    for cp in ucopies:
        cp.wait()
    for cp in mcopies:
        cp.wait()
    cp_ub.wait()
    cp_mb.wait()

    def group_body(g, carry):
        off = g * LANES
        acc = ub_v[pl.ds(off, LANES)] + mb_v[pl.ds(off, LANES)] + bias_vec
        for d in range(EMBED_DIM):
            acc = acc + (uvals_v[d, pl.ds(off, LANES)]
                         * mvals_v[d, pl.ds(off, LANES)])
        out_v[pl.ds(off, LANES)] = acc
        return carry

    lax.fori_loop(0, GROUPS, group_body, 0)

    pltpu.sync_copy(out_v, out_hbm.at[pl.ds(base, B_PER_W)])


@jax.jit
def _run(user, mission, uflat, mflat, ubias, mbias, bias):
    mesh = plsc.VectorSubcoreMesh(core_axis_name="c", subcore_axis_name="s")
    kfn = pl.kernel(
        _mf_kernel,
        out_type=jax.ShapeDtypeStruct((BATCH,), jnp.float32),
        mesh=mesh,
        compiler_params=pltpu.CompilerParams(needs_layout_passes=False,
                                             use_tc_tiling_on_sc=False),
        scratch_types=[
            pltpu.VMEM((B_PER_W,), jnp.int32),
            pltpu.VMEM((B_PER_W,), jnp.int32),
            pltpu.VMEM((EMBED_DIM, B_PER_W), jnp.int32),
            pltpu.VMEM((EMBED_DIM, B_PER_W), jnp.int32),
            pltpu.VMEM((EMBED_DIM, B_PER_W), jnp.float32),
            pltpu.VMEM((EMBED_DIM, B_PER_W), jnp.float32),
            pltpu.VMEM((B_PER_W,), jnp.float32),
            pltpu.VMEM((B_PER_W,), jnp.float32),
            pltpu.VMEM((LANES,), jnp.float32),
            pltpu.VMEM((B_PER_W,), jnp.float32),
            pltpu.SemaphoreType.DMA,
            pltpu.SemaphoreType.DMA,
            pltpu.SemaphoreType.DMA,
            pltpu.SemaphoreType.DMA,
        ],
    )
    return kfn(user, mission, uflat, mflat, ubias, mbias, bias)


def kernel(user, mission, user_embedding, mission_embedding, user_bias,
           mission_bias, bias):
    user = user.astype(jnp.int32)
    mission = mission.astype(jnp.int32)
    return _run(user, mission,
                user_embedding.T.reshape(-1), mission_embedding.T.reshape(-1),
                user_bias.reshape(-1), mission_bias.reshape(-1),
                bias.reshape(-1))


# speedup vs baseline: 4.8006x; 4.7484x over previous
import jax
import jax.numpy as jnp
from jax import lax
from jax.experimental import pallas as pl
from jax.experimental.pallas import tpu as pltpu
from jax.experimental.pallas import tpu_sc as plsc

BATCH = 16384
EMBED_DIM = 32
NUM_CORES = 2
NUM_SUBCORES = 16
LANES = 16
NUM_WORKERS = NUM_CORES * NUM_SUBCORES
B_PER_W = BATCH // NUM_WORKERS
GROUPS = B_PER_W // LANES


def _mf_kernel(user_hbm, mission_hbm, uemb_hbm, memb_hbm, ubias_hbm,
               mbias_hbm, bias_hbm, out_hbm,
               uidx_v, midx_v, urows_v, mrows_v, ub_v, mb_v, bias_v, out_v,
               sem_u, sem_m, sem_ub, sem_mb):
    wid = lax.axis_index("s") * NUM_CORES + lax.axis_index("c")
    base = wid * B_PER_W

    bias_v[...] = jnp.zeros((LANES,), jnp.float32)
    zidx = uidx_v
    zidx[pl.ds(0, LANES)] = jnp.zeros((LANES,), jnp.int32)
    pltpu.sync_copy(bias_hbm.at[zidx.at[pl.ds(0, LANES)]], bias_v)
    bias_vec = bias_v[...]

    pltpu.sync_copy(user_hbm.at[pl.ds(base, B_PER_W)], uidx_v)
    pltpu.sync_copy(mission_hbm.at[pl.ds(base, B_PER_W)], midx_v)

    cp_u = pltpu.async_copy(uemb_hbm.at[uidx_v], urows_v, sem_u)
    cp_m = pltpu.async_copy(memb_hbm.at[midx_v], mrows_v, sem_m)
    cp_ub = pltpu.async_copy(ubias_hbm.at[uidx_v], ub_v, sem_ub)
    cp_mb = pltpu.async_copy(mbias_hbm.at[midx_v], mb_v, sem_mb)
    cp_u.wait()
    cp_m.wait()
    cp_ub.wait()
    cp_mb.wait()

    lane_iota = lax.iota(jnp.int32, LANES)

    def group_body(g, carry):
        off = g * LANES
        rows = off + lane_iota
        acc = ub_v[pl.ds(off, LANES)] + mb_v[pl.ds(off, LANES)] + bias_vec
        for d in range(EMBED_DIM):
            col = jnp.full((LANES,), d, jnp.int32)
            uv = plsc.load_gather(urows_v, [rows, col])
            mv = plsc.load_gather(mrows_v, [rows, col])
            acc = acc + uv * mv
        out_v[pl.ds(off, LANES)] = acc
        return carry

    lax.fori_loop(0, GROUPS, group_body, 0)

    pltpu.sync_copy(out_v, out_hbm.at[pl.ds(base, B_PER_W)])


@jax.jit
def _run(user, mission, uemb, memb, ubias, mbias, bias):
    mesh = plsc.VectorSubcoreMesh(core_axis_name="c", subcore_axis_name="s")
    kfn = pl.kernel(
        _mf_kernel,
        out_type=jax.ShapeDtypeStruct((BATCH,), jnp.float32),
        mesh=mesh,
        compiler_params=pltpu.CompilerParams(needs_layout_passes=False,
                                             use_tc_tiling_on_sc=False),
        scratch_types=[
            pltpu.VMEM((B_PER_W,), jnp.int32),
            pltpu.VMEM((B_PER_W,), jnp.int32),
            pltpu.VMEM((B_PER_W, EMBED_DIM), jnp.float32),
            pltpu.VMEM((B_PER_W, EMBED_DIM), jnp.float32),
            pltpu.VMEM((B_PER_W,), jnp.float32),
            pltpu.VMEM((B_PER_W,), jnp.float32),
            pltpu.VMEM((LANES,), jnp.float32),
            pltpu.VMEM((B_PER_W,), jnp.float32),
            pltpu.SemaphoreType.DMA,
            pltpu.SemaphoreType.DMA,
            pltpu.SemaphoreType.DMA,
            pltpu.SemaphoreType.DMA,
        ],
    )
    return kfn(user, mission, uemb, memb, ubias, mbias, bias)


def kernel(user, mission, user_embedding, mission_embedding, user_bias,
           mission_bias, bias):
    user = user.astype(jnp.int32)
    mission = mission.astype(jnp.int32)
    return _run(user, mission, user_embedding, mission_embedding,
                user_bias.reshape(-1), mission_bias.reshape(-1),
                bias.reshape(-1))
